# Initial kernel scaffold; baseline (speedup 1.0000x reference)
import jax
import jax.numpy as jnp
from jax.experimental import pallas as pl


def kernel(x, edge_index, W1, b1, W2, b2):
    raise NotImplementedError("write your pallas kernel here")


# speedup vs baseline: 23.1278x; 23.1278x over previous
import functools

import jax
import jax.numpy as jnp
from jax import lax
from jax.experimental import pallas as pl
from jax.experimental.pallas import tpu as pltpu
from jax.experimental.pallas import tpu_sc as plsc

NC = 2
NS = 16
LANES = 16
CHUNK = 128
NBUF = 4
PGRP = 8
BR = 512
DH = 64


def _sc_mesh():
    return plsc.VectorSubcoreMesh(core_axis_name="c", subcore_axis_name="s")


def _degree_partials(dst_slab, npad, nchunk):
    rps = npad // NS
    half = nchunk // NC

    @functools.partial(
        pl.kernel,
        out_type=jax.ShapeDtypeStruct((NC, npad, LANES), jnp.float32),
        mesh=_sc_mesh(),
        scratch_types=[
            pltpu.VMEM((half, CHUNK), jnp.int32),
            pltpu.VMEM((CHUNK, LANES), jnp.float32),
            pltpu.VMEM((CHUNK, LANES), jnp.float32),
            pltpu.VMEM_SHARED((npad, LANES), jnp.float32),
            pltpu.SemaphoreType.DMA((PGRP,)),
        ],
    )
    def k(dst_hbm, out_hbm, dst_v, ones_v, zero_v, acc, sems):
        cid = lax.axis_index("c")
        sid = lax.axis_index("s")

        @pl.loop(0, CHUNK)
        def _(r):
            ones_v[r, :] = jnp.full((LANES,), 1.0, jnp.float32)
            zero_v[r, :] = jnp.zeros((LANES,), jnp.float32)

        pltpu.sync_copy(dst_hbm.at[sid].at[pl.ds(cid * half, half)], dst_v)

        @pl.loop(0, rps // CHUNK)
        def _(i):
            pltpu.sync_copy(zero_v, acc.at[pl.ds(sid * rps + i * CHUNK, CHUNK)])

        plsc.subcore_barrier()

        @pl.loop(0, half, step=PGRP)
        def _(c0):
            descs = []
            for b in range(PGRP):
                descs.append(
                    pltpu.async_copy(
                        ones_v, acc.at[dst_v.at[c0 + b]], sems.at[b], add=True
                    )
                )
            for d in descs:
                d.wait()

        plsc.subcore_barrier()
        pltpu.sync_copy(
            acc.at[pl.ds(sid * rps, rps)],
            out_hbm.at[cid].at[pl.ds(sid * rps, rps)],
        )

    return k(dst_slab)


def _aggregate(gs, src_slab, dst_slab, npad, nchunk):
    rps = npad // NS

    @functools.partial(
        pl.kernel,
        out_type=jax.ShapeDtypeStruct((NC, npad, DH), jnp.float32),
        mesh=_sc_mesh(),
        compiler_params=pltpu.CompilerParams(use_tc_tiling_on_sc=False),
        scratch_types=[
            pltpu.VMEM((nchunk, CHUNK), jnp.int32),
            pltpu.VMEM((nchunk, CHUNK), jnp.int32),
            pltpu.VMEM((NBUF, CHUNK, DH), jnp.float32),
            pltpu.VMEM_SHARED((npad, DH), jnp.float32),
            pltpu.SemaphoreType.DMA((NBUF,)),
            pltpu.SemaphoreType.DMA((NBUF,)),
        ],
    )
    def k(gs_hbm, src_hbm, dst_hbm, out_hbm, src_v, dst_v, rows, acc, gsems, ssems):
        cid = lax.axis_index("c")
        sid = lax.axis_index("s")

        pltpu.sync_copy(src_hbm.at[sid], src_v)
        pltpu.sync_copy(dst_hbm.at[sid], dst_v)

        @pl.loop(0, CHUNK)
        def _(r):
            for j in range(DH // LANES):
                rows[0, r, pl.ds(j * LANES, LANES)] = jnp.zeros(
                    (LANES,), jnp.float32
                )

        @pl.loop(0, rps // CHUNK)
        def _(i):
            pltpu.sync_copy(
                rows.at[0], acc.at[pl.ds(sid * rps + i * CHUNK, CHUNK)]
            )

        plsc.subcore_barrier()

        @pl.loop(0, nchunk, step=NBUF)
        def _(c0):
            gds = []
            for b in range(NBUF):
                gds.append(
                    pltpu.async_copy(
                        gs_hbm.at[cid].at[src_v.at[c0 + b]],
                        rows.at[b],
                        gsems.at[b],
                    )
                )
            sds = []
            for b in range(NBUF):
                gds[b].wait()
                sds.append(
                    pltpu.async_copy(
                        rows.at[b],
                        acc.at[dst_v.at[c0 + b]],
                        ssems.at[b],
                        add=True,
                    )
                )
            for d in sds:
                d.wait()

        plsc.subcore_barrier()
        pltpu.sync_copy(
            acc.at[pl.ds(sid * rps, rps)],
            out_hbm.at[cid].at[pl.ds(sid * rps, rps)],
        )

    return k(gs, src_slab, dst_slab)


def _dinv_block(degp_blk):
    deg = degp_blk[0, :, 0:1] + degp_blk[1, :, 0:1] + 1.0
    return lax.rsqrt(deg)


def _split_ref(o_ref, g):
    o_ref[0] = g[:, :DH]
    o_ref[1] = g[:, DH:]


def _g1_tc(x, W1, degp, npad):
    n, d_in = x.shape
    dm = W1.shape[1]

    def body(x_ref, w_ref, degp_ref, g_ref):
        h = jnp.dot(x_ref[...], w_ref[...], preferred_element_type=jnp.float32)
        _split_ref(g_ref, h * _dinv_block(degp_ref[...]))

    return pl.pallas_call(
        body,
        grid=(npad // BR,),
        in_specs=[
            pl.BlockSpec((BR, d_in), lambda i: (i, 0)),
            pl.BlockSpec((d_in, dm), lambda i: (0, 0)),
            pl.BlockSpec((NC, BR, LANES), lambda i: (0, i, 0)),
        ],
        out_specs=pl.BlockSpec((NC, BR, DH), lambda i: (0, i, 0)),
        out_shape=jax.ShapeDtypeStruct((NC, npad, DH), jnp.float32),
    )(x, W1, degp)


def _g2_tc(agg, gs1, degp, b1, W2, npad):
    dm = W2.shape[1]

    def body(a_ref, g1_ref, degp_ref, b1_ref, w_ref, g2_ref):
        dinv = _dinv_block(degp_ref[...])
        s = jnp.concatenate(
            [a_ref[0] + g1_ref[0], a_ref[1] + g1_ref[1]], axis=-1
        )
        t = jnp.maximum(s * dinv + b1_ref[...], 0.0)
        h = jnp.dot(t, w_ref[...], preferred_element_type=jnp.float32)
        _split_ref(g2_ref, h * dinv)

    return pl.pallas_call(
        body,
        grid=(npad // BR,),
        in_specs=[
            pl.BlockSpec((NC, BR, DH), lambda i: (0, i, 0)),
            pl.BlockSpec((NC, BR, DH), lambda i: (0, i, 0)),
            pl.BlockSpec((NC, BR, LANES), lambda i: (0, i, 0)),
            pl.BlockSpec((1, dm), lambda i: (0, 0)),
            pl.BlockSpec((dm, dm), lambda i: (0, 0)),
        ],
        out_specs=pl.BlockSpec((NC, BR, DH), lambda i: (0, i, 0)),
        out_shape=jax.ShapeDtypeStruct((NC, npad, DH), jnp.float32),
    )(agg, gs1, degp, b1, W2)


def _final_tc(agg, gs2, degp, b2, n):
    npad = gs2.shape[1]
    dm = NC * DH

    def body(a_ref, g2_ref, degp_ref, b2_ref, o_ref):
        dinv = _dinv_block(degp_ref[...])
        s = jnp.concatenate(
            [a_ref[0] + g2_ref[0], a_ref[1] + g2_ref[1]], axis=-1
        )
        o_ref[...] = s * dinv + b2_ref[...]

    return pl.pallas_call(
        body,
        grid=(npad // BR,),
        in_specs=[
            pl.BlockSpec((NC, BR, DH), lambda i: (0, i, 0)),
            pl.BlockSpec((NC, BR, DH), lambda i: (0, i, 0)),
            pl.BlockSpec((NC, BR, LANES), lambda i: (0, i, 0)),
            pl.BlockSpec((1, dm), lambda i: (0, 0)),
        ],
        out_specs=pl.BlockSpec((BR, dm), lambda i: (i, 0)),
        out_shape=jax.ShapeDtypeStruct((n, dm), jnp.float32),
    )(agg, gs2, degp, b2)


def kernel(x, edge_index, W1, b1, W2, b2):
    n, d_in = x.shape
    e = edge_index.shape[1]

    npad = -(-n // (NS * CHUNK)) * (NS * CHUNK)
    per_w = -(-e // NS)
    nchunk = -(-per_w // CHUNK)
    nchunk = -(-nchunk // (NC * PGRP)) * (NC * PGRP)
    epad = NS * nchunk * CHUNK

    src = edge_index[0]
    dst = edge_index[1]
    pad = epad - e
    ar = jnp.arange(pad, dtype=jnp.int32)
    src_slab = jnp.concatenate([src, (ar * 37) % n]).reshape(NS, nchunk, CHUNK)
    dst_slab = jnp.concatenate([dst, n + ar % (npad - n)]).reshape(
        NS, nchunk, CHUNK
    )

    degp = _degree_partials(dst_slab, npad, nchunk)
    gs1 = _g1_tc(x, W1, degp, npad)
    agg1 = _aggregate(gs1, src_slab, dst_slab, npad, nchunk)
    gs2 = _g2_tc(agg1, gs1, degp, b1.reshape(1, -1), W2, npad)
    agg2 = _aggregate(gs2, src_slab, dst_slab, npad, nchunk)
    return _final_tc(agg2, gs2, degp, b2.reshape(1, -1), n)


# speedup vs baseline: 25.7631x; 1.1139x over previous
import functools

import jax
import jax.numpy as jnp
from jax import lax
from jax.experimental import pallas as pl
from jax.experimental.pallas import tpu as pltpu
from jax.experimental.pallas import tpu_sc as plsc

NC = 2
NS = 16
LANES = 16
CHUNK = 128
NBUF = 2
PGRP = 8
BR = 512
DH = 64


def _sc_mesh():
    return plsc.VectorSubcoreMesh(core_axis_name="c", subcore_axis_name="s")


def _degree_partials(dst_slab, npad, nchunk):
    rps = npad // NS
    half = nchunk // NC

    @functools.partial(
        pl.kernel,
        out_type=jax.ShapeDtypeStruct((NC, npad, LANES), jnp.float32),
        mesh=_sc_mesh(),
        scratch_types=[
            pltpu.VMEM((half, CHUNK), jnp.int32),
            pltpu.VMEM((CHUNK, LANES), jnp.float32),
            pltpu.VMEM((CHUNK, LANES), jnp.float32),
            pltpu.VMEM_SHARED((npad, LANES), jnp.float32),
            pltpu.SemaphoreType.DMA((PGRP,)),
        ],
    )
    def k(dst_hbm, out_hbm, dst_v, ones_v, zero_v, acc, sems):
        cid = lax.axis_index("c")
        sid = lax.axis_index("s")

        @pl.loop(0, CHUNK)
        def _(r):
            ones_v[r, :] = jnp.full((LANES,), 1.0, jnp.float32)
            zero_v[r, :] = jnp.zeros((LANES,), jnp.float32)

        pltpu.sync_copy(dst_hbm.at[sid].at[pl.ds(cid * half, half)], dst_v)

        @pl.loop(0, rps // CHUNK)
        def _(i):
            pltpu.sync_copy(zero_v, acc.at[pl.ds(sid * rps + i * CHUNK, CHUNK)])

        plsc.subcore_barrier()

        @pl.loop(0, half, step=PGRP)
        def _(c0):
            descs = []
            for b in range(PGRP):
                descs.append(
                    pltpu.async_copy(
                        ones_v, acc.at[dst_v.at[c0 + b]], sems.at[b], add=True
                    )
                )
            for d in descs:
                d.wait()

        plsc.subcore_barrier()
        pltpu.sync_copy(
            acc.at[pl.ds(sid * rps, rps)],
            out_hbm.at[cid].at[pl.ds(sid * rps, rps)],
        )

    return k(dst_slab)


def _aggregate(gs, src_slab, dst_slab, npad, nchunk):
    rps = npad // NS

    @functools.partial(
        pl.kernel,
        out_type=jax.ShapeDtypeStruct((NC, npad, DH), jnp.float32),
        mesh=_sc_mesh(),
        compiler_params=pltpu.CompilerParams(use_tc_tiling_on_sc=False),
        scratch_types=[
            pltpu.VMEM((nchunk, CHUNK), jnp.int32),
            pltpu.VMEM((nchunk, CHUNK), jnp.int32),
            pltpu.VMEM((NBUF, CHUNK, DH), jnp.float32),
            pltpu.VMEM((NBUF, CHUNK, DH), jnp.float32),
            pltpu.VMEM_SHARED((npad, DH), jnp.float32),
            pltpu.SemaphoreType.DMA((NBUF,)),
            pltpu.SemaphoreType.DMA((NBUF,)),
            pltpu.SemaphoreType.DMA((NBUF,)),
            pltpu.SemaphoreType.DMA((NBUF,)),
        ],
    )
    def k(gs_hbm, src_hbm, dst_hbm, out_hbm, src_v, dst_v, rows0, rows1, acc,
          gsems0, gsems1, ssems0, ssems1):
        rows_s = (rows0, rows1)
        gsems_s = (gsems0, gsems1)
        ssems_s = (ssems0, ssems1)
        cid = lax.axis_index("c")
        sid = lax.axis_index("s")

        pltpu.sync_copy(src_hbm.at[sid], src_v)
        pltpu.sync_copy(dst_hbm.at[sid], dst_v)

        @pl.loop(0, CHUNK)
        def _(r):
            for j in range(DH // LANES):
                rows0[0, r, pl.ds(j * LANES, LANES)] = jnp.zeros(
                    (LANES,), jnp.float32
                )
                rows1[0, r, pl.ds(j * LANES, LANES)] = jnp.zeros(
                    (LANES,), jnp.float32
                )

        @pl.loop(0, rps // CHUNK)
        def _(i):
            pltpu.sync_copy(
                rows0.at[0], acc.at[pl.ds(sid * rps + i * CHUNK, CHUNK)]
            )

        plsc.subcore_barrier()

        def gath(c0, s, b):
            return pltpu.make_async_copy(
                gs_hbm.at[cid].at[src_v.at[c0 + b]],
                rows_s[s].at[b],
                gsems_s[s].at[b],
            )

        def scat(c0, s, b):
            return pltpu.make_async_copy(
                rows_s[s].at[b],
                acc.at[dst_v.at[c0 + b]],
                ssems_s[s].at[b],
            )

        ngrp = nchunk // NBUF

        for b in range(NBUF):
            gath(0, 0, b).start()
        for b in range(NBUF):
            gath(0, 0, b).wait()
            scat(0, 0, b).start(add=True)
        for b in range(NBUF):
            gath(NBUF, 1, b).start()

        @pl.loop(1, ngrp - 2, step=2)
        def _(k):
            c1 = k * NBUF
            for s in (1, 0):
                for b in range(NBUF):
                    gath(c1, s, b).wait()
                    scat(c1, s, b).start(add=True)
                for b in range(NBUF):
                    scat(c1 - NBUF, 1 - s, b).wait()
                for b in range(NBUF):
                    gath(c1 + NBUF, 1 - s, b).start()
                c1 += NBUF

        cl = (ngrp - 1) * NBUF
        for b in range(NBUF):
            gath(cl, 1, b).wait()
            scat(cl, 1, b).start(add=True)
        for b in range(NBUF):
            scat(cl - NBUF, 0, b).wait()
        for b in range(NBUF):
            scat(cl, 1, b).wait()

        plsc.subcore_barrier()
        pltpu.sync_copy(
            acc.at[pl.ds(sid * rps, rps)],
            out_hbm.at[cid].at[pl.ds(sid * rps, rps)],
        )

    return k(gs, src_slab, dst_slab)


def _dinv_block(degp_blk):
    deg = degp_blk[0, :, 0:1] + degp_blk[1, :, 0:1] + 1.0
    return lax.rsqrt(deg)


def _split_ref(o_ref, g):
    o_ref[0] = g[:, :DH]
    o_ref[1] = g[:, DH:]


def _g1_tc(x, W1, degp, npad):
    n, d_in = x.shape
    dm = W1.shape[1]

    def body(x_ref, w_ref, degp_ref, g_ref):
        h = jnp.dot(x_ref[...], w_ref[...], preferred_element_type=jnp.float32)
        _split_ref(g_ref, h * _dinv_block(degp_ref[...]))

    return pl.pallas_call(
        body,
        grid=(npad // BR,),
        in_specs=[
            pl.BlockSpec((BR, d_in), lambda i: (i, 0)),
            pl.BlockSpec((d_in, dm), lambda i: (0, 0)),
            pl.BlockSpec((NC, BR, LANES), lambda i: (0, i, 0)),
        ],
        out_specs=pl.BlockSpec((NC, BR, DH), lambda i: (0, i, 0)),
        out_shape=jax.ShapeDtypeStruct((NC, npad, DH), jnp.float32),
    )(x, W1, degp)


def _g2_tc(agg, gs1, degp, b1, W2, npad):
    dm = W2.shape[1]

    def body(a_ref, g1_ref, degp_ref, b1_ref, w_ref, g2_ref):
        dinv = _dinv_block(degp_ref[...])
        s = jnp.concatenate(
            [a_ref[0] + g1_ref[0], a_ref[1] + g1_ref[1]], axis=-1
        )
        t = jnp.maximum(s * dinv + b1_ref[...], 0.0)
        h = jnp.dot(t, w_ref[...], preferred_element_type=jnp.float32)
        _split_ref(g2_ref, h * dinv)

    return pl.pallas_call(
        body,
        grid=(npad // BR,),
        in_specs=[
            pl.BlockSpec((NC, BR, DH), lambda i: (0, i, 0)),
            pl.BlockSpec((NC, BR, DH), lambda i: (0, i, 0)),
            pl.BlockSpec((NC, BR, LANES), lambda i: (0, i, 0)),
            pl.BlockSpec((1, dm), lambda i: (0, 0)),
            pl.BlockSpec((dm, dm), lambda i: (0, 0)),
        ],
        out_specs=pl.BlockSpec((NC, BR, DH), lambda i: (0, i, 0)),
        out_shape=jax.ShapeDtypeStruct((NC, npad, DH), jnp.float32),
    )(agg, gs1, degp, b1, W2)


def _final_tc(agg, gs2, degp, b2, n):
    npad = gs2.shape[1]
    dm = NC * DH

    def body(a_ref, g2_ref, degp_ref, b2_ref, o_ref):
        dinv = _dinv_block(degp_ref[...])
        s = jnp.concatenate(
            [a_ref[0] + g2_ref[0], a_ref[1] + g2_ref[1]], axis=-1
        )
        o_ref[...] = s * dinv + b2_ref[...]

    return pl.pallas_call(
        body,
        grid=(npad // BR,),
        in_specs=[
            pl.BlockSpec((NC, BR, DH), lambda i: (0, i, 0)),
            pl.BlockSpec((NC, BR, DH), lambda i: (0, i, 0)),
            pl.BlockSpec((NC, BR, LANES), lambda i: (0, i, 0)),
            pl.BlockSpec((1, dm), lambda i: (0, 0)),
        ],
        out_specs=pl.BlockSpec((BR, dm), lambda i: (i, 0)),
        out_shape=jax.ShapeDtypeStruct((n, dm), jnp.float32),
    )(agg, gs2, degp, b2)


def kernel(x, edge_index, W1, b1, W2, b2):
    n, d_in = x.shape
    e = edge_index.shape[1]

    npad = -(-n // (NS * CHUNK)) * (NS * CHUNK)
    per_w = -(-e // NS)
    nchunk = -(-per_w // CHUNK)
    nchunk = -(-nchunk // (NC * PGRP)) * (NC * PGRP)
    epad = NS * nchunk * CHUNK

    src = edge_index[0]
    dst = edge_index[1]
    pad = epad - e
    ar = jnp.arange(pad, dtype=jnp.int32)
    src_slab = jnp.concatenate([src, (ar * 37) % n]).reshape(NS, nchunk, CHUNK)
    dst_slab = jnp.concatenate([dst, n + ar % (npad - n)]).reshape(
        NS, nchunk, CHUNK
    )

    degp = _degree_partials(dst_slab, npad, nchunk)
    gs1 = _g1_tc(x, W1, degp, npad)
    agg1 = _aggregate(gs1, src_slab, dst_slab, npad, nchunk)
    gs2 = _g2_tc(agg1, gs1, degp, b1.reshape(1, -1), W2, npad)
    agg2 = _aggregate(gs2, src_slab, dst_slab, npad, nchunk)
    return _final_tc(agg2, gs2, degp, b2.reshape(1, -1), n)
